# baseline (speedup 1.0000x reference)
import math

import jax
import jax.numpy as jnp
from jax import lax
from jax.experimental import pallas as pl
from jax.experimental.pallas import tpu as pltpu
from jax.experimental.pallas import tpu_sc as plsc

_SIZE = 100000
_N = 2048
_PAD = 0
_SMOOTH = 0.1
_EPS = _SMOOTH / (_SIZE - 2)
_CONF = 1.0 - _SMOOTH
_K = (_SIZE - 2) * _EPS * math.log(_EPS) + _CONF * math.log(_CONF)

_L = 16
_NC = 2

_CB = 2048
_GRID = (_SIZE + _CB - 1) // _CB
_LAST_ROWS = _SIZE - (_GRID - 1) * _CB
_SUB = 8


def _accum_chunks(acc_ref, y_ref, t, base, nchunks):
    for k in range(nchunks):
        yc = y_ref[k * _SUB:(k + 1) * _SUB, :]
        rows = (base + k * _SUB) + lax.broadcasted_iota(
            jnp.int32, yc.shape, 0)
        coef = jnp.where(rows == t, -_CONF, -_EPS)
        acc_ref[...] += coef * yc


def _tc_body(y_ref, t_ref, p_ref, acc_ref):
    i = pl.program_id(0)
    t = t_ref[...]

    @pl.when(i == 0)
    def _init():
        acc_ref[...] = jnp.zeros_like(acc_ref)

    @pl.when(i < _GRID - 1)
    def _interior():
        _accum_chunks(acc_ref, y_ref, t, i * _CB, _CB // _SUB)

    @pl.when(i == 0)
    def _fix_pad_col():
        acc_ref[0:1, :] += _EPS * y_ref[0:1, :]

    @pl.when(i == _GRID - 1)
    def _last():
        _accum_chunks(acc_ref, y_ref, t, i * _CB, _LAST_ROWS // _SUB)
        p_ref[...] = jnp.sum(acc_ref[...], axis=0, keepdims=True)


def _sc_body(p_hbm, t_hbm, out_hbm, p_v, t_v, out_v):
    wid = lax.axis_index("s") * _NC + lax.axis_index("c")

    @pl.when(wid == 0)
    def _combine():
        pltpu.sync_copy(p_hbm, p_v)
        pltpu.sync_copy(t_hbm, t_v)

        def body(k, acc):
            sl = pl.ds(k * _L, _L)
            m = jnp.where(t_v[sl] == _PAD, 0.0, 1.0)
            return acc + m * (_K + p_v[sl])

        out_v[...] = lax.fori_loop(0, _N // _L, body,
                                   jnp.zeros((_L,), jnp.float32))
        pltpu.sync_copy(out_v, out_hbm)


def _make_sc_call():
    return pl.kernel(
        _sc_body,
        out_type=jax.ShapeDtypeStruct((_L,), jnp.float32),
        mesh=plsc.VectorSubcoreMesh(core_axis_name="c", subcore_axis_name="s"),
        scratch_types=[
            pltpu.VMEM((_N,), jnp.float32),
            pltpu.VMEM((_N,), jnp.int32),
            pltpu.VMEM((_L,), jnp.float32),
        ],
    )


def kernel(x, target):
    y = x.T
    t2d = target.astype(jnp.int32).reshape(1, _N)
    p = pl.pallas_call(
        _tc_body,
        grid=(_GRID,),
        in_specs=[
            pl.BlockSpec((_CB, _N), lambda i: (i, 0)),
            pl.BlockSpec((1, _N), lambda i: (0, 0)),
        ],
        out_specs=pl.BlockSpec((1, _N), lambda i: (0, 0)),
        out_shape=jax.ShapeDtypeStruct((1, _N), jnp.float32),
        scratch_shapes=[pltpu.VMEM((_SUB, _N), jnp.float32)],
    )(y, t2d)
    out = _make_sc_call()(p.reshape(-1), target.astype(jnp.int32))
    return jnp.sum(out)


# speedup vs baseline: 1.0025x; 1.0025x over previous
import math

import jax
import jax.numpy as jnp
from jax import lax
from jax.experimental import pallas as pl
from jax.experimental.pallas import tpu as pltpu
from jax.experimental.pallas import tpu_sc as plsc

_SIZE = 100000
_N = 2048
_PAD = 0
_SMOOTH = 0.1
_EPS = _SMOOTH / (_SIZE - 2)
_CONF = 1.0 - _SMOOTH
_K = (_SIZE - 2) * _EPS * math.log(_EPS) + _CONF * math.log(_CONF)

_L = 16
_NC = 2

_CB = 1024
_GRID = (_SIZE + _CB - 1) // _CB
_LAST_ROWS = _SIZE - (_GRID - 1) * _CB
_SUB = 8


def _accum_chunks(acc_ref, y_ref, t, base, nchunks):
    for k in range(nchunks):
        yc = y_ref[k * _SUB:(k + 1) * _SUB, :]
        rows = (base + k * _SUB) + lax.broadcasted_iota(
            jnp.int32, yc.shape, 0)
        coef = jnp.where(rows == t, -_CONF, -_EPS)
        acc_ref[...] += coef * yc


def _tc_body(y_ref, t_ref, p_ref, acc_ref):
    i = pl.program_id(0)
    t = t_ref[...]

    @pl.when(i == 0)
    def _init():
        acc_ref[...] = jnp.zeros_like(acc_ref)

    @pl.when(i < _GRID - 1)
    def _interior():
        _accum_chunks(acc_ref, y_ref, t, i * _CB, _CB // _SUB)

    @pl.when(i == 0)
    def _fix_pad_col():
        acc_ref[0:1, :] += _EPS * y_ref[0:1, :]

    @pl.when(i == _GRID - 1)
    def _last():
        _accum_chunks(acc_ref, y_ref, t, i * _CB, _LAST_ROWS // _SUB)
        p_ref[...] = jnp.sum(acc_ref[...], axis=0, keepdims=True)


def _sc_body(p_hbm, t_hbm, out_hbm, p_v, t_v, out_v):
    wid = lax.axis_index("s") * _NC + lax.axis_index("c")

    @pl.when(wid == 0)
    def _combine():
        pltpu.sync_copy(p_hbm, p_v)
        pltpu.sync_copy(t_hbm, t_v)

        def body(k, acc):
            sl = pl.ds(k * _L, _L)
            m = jnp.where(t_v[sl] == _PAD, 0.0, 1.0)
            return acc + m * (_K + p_v[sl])

        out_v[...] = lax.fori_loop(0, _N // _L, body,
                                   jnp.zeros((_L,), jnp.float32))
        pltpu.sync_copy(out_v, out_hbm)


def _make_sc_call():
    return pl.kernel(
        _sc_body,
        out_type=jax.ShapeDtypeStruct((_L,), jnp.float32),
        mesh=plsc.VectorSubcoreMesh(core_axis_name="c", subcore_axis_name="s"),
        scratch_types=[
            pltpu.VMEM((_N,), jnp.float32),
            pltpu.VMEM((_N,), jnp.int32),
            pltpu.VMEM((_L,), jnp.float32),
        ],
    )


def kernel(x, target):
    y = x.T
    t2d = target.astype(jnp.int32).reshape(1, _N)
    p = pl.pallas_call(
        _tc_body,
        grid=(_GRID,),
        in_specs=[
            pl.BlockSpec((_CB, _N), lambda i: (i, 0)),
            pl.BlockSpec((1, _N), lambda i: (0, 0)),
        ],
        out_specs=pl.BlockSpec((1, _N), lambda i: (0, 0)),
        out_shape=jax.ShapeDtypeStruct((1, _N), jnp.float32),
        scratch_shapes=[pltpu.VMEM((_SUB, _N), jnp.float32)],
    )(y, t2d)
    out = _make_sc_call()(p.reshape(-1), target.astype(jnp.int32))
    return jnp.sum(out)
